# baseline (speedup 1.0000x reference)
import math

import jax
import jax.numpy as jnp
from jax.experimental import pallas as pl
from jax.experimental.pallas import tpu as pltpu


def _dft_mats(N, l):
    n = jnp.arange(N, dtype=jnp.float32)[:, None]
    m = jnp.arange(l, dtype=jnp.float32)[None, :]
    ang = 2.0 * math.pi * n * m / float(N)
    cosm, sinm = jnp.cos(ang), jnp.sin(ang)
    wgt = jnp.where((jnp.arange(l) == 0) |
                    ((N % 2 == 0) & (jnp.arange(l) == N // 2)),
                    1.0, 2.0).astype(jnp.float32) / float(N)
    ffwd = jnp.concatenate([cosm, -sinm], axis=1)
    finv = jnp.concatenate([wgt[:, None] * cosm.T,
                            -wgt[:, None] * sinm.T], axis=0)
    return ffwd, finv


def _mix_weights(weights_r, weights_i, l):
    wr = jnp.transpose(weights_r[:, :, :l], (2, 0, 1))
    wi = jnp.transpose(weights_i[:, :, :l], (2, 0, 1))
    return jnp.concatenate(
        [jnp.concatenate([wr, wi], axis=-1),
         jnp.concatenate([-wi, wr], axis=-1)], axis=0)


def _make_body(TB, N, D, l):
    l2 = 2 * l

    def body(x_ref, ffwd_ref, wcat_ref, finv_ref, o_ref):
        xb = x_ref[...].astype(jnp.bfloat16)
        xt = jnp.swapaxes(xb, 1, 2).reshape(TB * D, N)
        spec = jnp.dot(xt, ffwd_ref[...],
                       preferred_element_type=jnp.float32)
        spec_m = spec.astype(jnp.bfloat16).T.reshape(l2, TB, D)
        p = jnp.einsum('mbi,mio->mbo', spec_m, wcat_ref[...],
                       preferred_element_type=jnp.float32)
        y = p[:l] + p[l:]
        ys = jnp.concatenate([y[:, :, :D], y[:, :, D:]], axis=0)
        yk = ys.astype(jnp.bfloat16).reshape(l2, TB * D).T
        out = jnp.dot(yk, finv_ref[...],
                      preferred_element_type=jnp.float32)
        o_ref[...] = jnp.swapaxes(out.reshape(TB, D, N), 1, 2)

    return body


def kernel(x, weights_r, weights_i):
    B, N, c, k = x.shape
    D = c * k
    modes1 = weights_r.shape[-1]
    l = min(modes1, N // 2 + 1)
    l2 = 2 * l

    x3 = x.reshape(B, N, D)
    ffwd, finv = _dft_mats(N, l)
    wcat = _mix_weights(weights_r, weights_i, l)
    ffwd = ffwd.astype(jnp.bfloat16)
    finv = finv.astype(jnp.bfloat16)
    wcat = wcat.astype(jnp.bfloat16)

    TB = 128
    while B % TB:
        TB //= 2
    grid = (B // TB,)

    flops = int(2 * B * D * N * l2 + 2 * B * l2 * D * 2 * D
                + 2 * B * D * l2 * N)
    bytes_accessed = int(4 * 2 * B * N * D
                         + 2 * (N * l2 + l2 * N + l2 * D * 2 * D))

    out = pl.pallas_call(
        _make_body(TB, N, D, l),
        out_shape=jax.ShapeDtypeStruct((B, N, D), jnp.float32),
        grid=grid,
        in_specs=[
            pl.BlockSpec((TB, N, D), lambda b: (b, 0, 0)),
            pl.BlockSpec((N, l2), lambda b: (0, 0),
                         pipeline_mode=pl.Buffered(1)),
            pl.BlockSpec((l2, D, 2 * D), lambda b: (0, 0, 0),
                         pipeline_mode=pl.Buffered(1)),
            pl.BlockSpec((l2, N), lambda b: (0, 0),
                         pipeline_mode=pl.Buffered(1)),
        ],
        out_specs=pl.BlockSpec((TB, N, D), lambda b: (b, 0, 0)),
        compiler_params=pltpu.CompilerParams(
            dimension_semantics=("parallel",),
            vmem_limit_bytes=100 * 2 ** 20),
        cost_estimate=pl.CostEstimate(
            flops=flops, transcendentals=0, bytes_accessed=bytes_accessed),
    )(x3, ffwd, wcat, finv)

    return out.reshape(B, N, c, k)


# speedup vs baseline: 4.6327x; 4.6327x over previous
import math

import jax
import jax.numpy as jnp
from jax.experimental import pallas as pl
from jax.experimental.pallas import tpu as pltpu


def _dft_mats(N, l):
    n = jnp.arange(N, dtype=jnp.float32)[:, None]
    m = jnp.arange(l, dtype=jnp.float32)[None, :]
    ang = 2.0 * math.pi * n * m / float(N)
    cosm, sinm = jnp.cos(ang), jnp.sin(ang)
    wgt = jnp.where((jnp.arange(l) == 0) |
                    ((N % 2 == 0) & (jnp.arange(l) == N // 2)),
                    1.0, 2.0).astype(jnp.float32) / float(N)
    ffwd = jnp.concatenate([cosm, -sinm], axis=1)
    finv = jnp.concatenate([wgt[:, None] * cosm.T,
                            -wgt[:, None] * sinm.T], axis=0)
    return ffwd, finv


def _mix_weights(weights_r, weights_i, l):
    wr = jnp.transpose(weights_r[:, :, :l], (2, 0, 1))
    wi = jnp.transpose(weights_i[:, :, :l], (2, 0, 1))
    return jnp.concatenate(
        [jnp.concatenate([wr, wi], axis=-1),
         jnp.concatenate([-wi, wr], axis=-1)], axis=0)


def _make_body(TB, D, l):
    l2 = 2 * l

    def body(x_ref, ffwd_ref, wcat_ref, finv_ref, o_ref):
        xt = x_ref[...].astype(jnp.bfloat16)
        spec_m = jax.lax.dot_general(
            ffwd_ref[...], xt, (((0,), (1,)), ((), ())),
            preferred_element_type=jnp.float32)
        spec_m = spec_m.astype(jnp.bfloat16).reshape(l2, TB, D)
        p = jnp.einsum('mbi,mio->mbo', spec_m, wcat_ref[...],
                       preferred_element_type=jnp.float32)
        y = p[:l] + p[l:]
        ys = jnp.concatenate([y[:, :, :D], y[:, :, D:]], axis=0)
        out = jax.lax.dot_general(
            ys.astype(jnp.bfloat16), finv_ref[...],
            (((0,), (0,)), ((), ())),
            preferred_element_type=jnp.float32)
        o_ref[...] = out.reshape(TB * D, out.shape[-1])

    return body


def kernel(x, weights_r, weights_i):
    B, N, c, k = x.shape
    D = c * k
    modes1 = weights_r.shape[-1]
    l = min(modes1, N // 2 + 1)
    l2 = 2 * l

    x_flat = jnp.transpose(x.reshape(B, N, D), (0, 2, 1)).reshape(B * D, N)

    ffwd, finv = _dft_mats(N, l)
    wcat = _mix_weights(weights_r, weights_i, l)
    ffwd = ffwd.astype(jnp.bfloat16)
    finv = finv.astype(jnp.bfloat16)
    wcat = wcat.astype(jnp.bfloat16)

    TB = 256
    while B % TB:
        TB //= 2
    grid = (B // TB,)

    flops = int(2 * B * D * N * l2 + 2 * B * l2 * D * 2 * D
                + 2 * B * D * l2 * N)
    bytes_accessed = int(4 * 2 * B * N * D
                         + 2 * (N * l2 + l2 * N + l2 * D * 2 * D))

    out_flat = pl.pallas_call(
        _make_body(TB, D, l),
        out_shape=jax.ShapeDtypeStruct((B * D, N), jnp.float32),
        grid=grid,
        in_specs=[
            pl.BlockSpec((TB * D, N), lambda b: (b, 0)),
            pl.BlockSpec((N, l2), lambda b: (0, 0),
                         pipeline_mode=pl.Buffered(1)),
            pl.BlockSpec((l2, D, 2 * D), lambda b: (0, 0, 0),
                         pipeline_mode=pl.Buffered(1)),
            pl.BlockSpec((l2, N), lambda b: (0, 0),
                         pipeline_mode=pl.Buffered(1)),
        ],
        out_specs=pl.BlockSpec((TB * D, N), lambda b: (b, 0)),
        compiler_params=pltpu.CompilerParams(
            dimension_semantics=("parallel",),
            vmem_limit_bytes=100 * 2 ** 20),
        cost_estimate=pl.CostEstimate(
            flops=flops, transcendentals=0, bytes_accessed=bytes_accessed),
    )(x_flat, ffwd, wcat, finv)

    return jnp.transpose(out_flat.reshape(B, D, N), (0, 2, 1)).reshape(B, N, c, k)
